# baseline (speedup 1.0000x reference)
import jax
import jax.numpy as jnp
from jax.experimental import pallas as pl
from jax.experimental.pallas import tpu as pltpu

_B = 16384
_CHUNK = 1024
_NCHUNK = _B // _CHUNK
_NBUF = 4
_VIEW = (128, 256, 512)
_NW = 18


def _dot(a, b):
    return jnp.dot(a, b, preferred_element_type=jnp.float32)


_BF = jnp.bfloat16


def _body(*refs):
    xs = refs[0:3]
    whbm = refs[3:3 + 3 * _NW]
    outs = refs[3 + 3 * _NW:12 + 3 * _NW]
    wrefs = refs[12 + 3 * _NW:12 + 6 * _NW]
    (xb0, xb1, xb2, rxb0, rxb1, rxb2, zcb0, zcb1, zcb2, zsb0, zsb1, zsb2,
     wa0, wa1, wa2, wb0, wb1, wb2, wc0, wc1, wc2,
     wt0, wt1, wt2, wd10, wd11, wd12, wd20, wd21, wd22,
     sin, srx, sz, swt) = refs[12 + 6 * _NW:]
    wtbufs = (wt0, wt1, wt2)
    wd1bufs = (wd10, wd11, wd12)
    wd2bufs = (wd20, wd21, wd22)
    wabufs = (wa0, wa1, wa2)
    wbbufs = (wb0, wb1, wb2)
    wcbufs = (wc0, wc1, wc2)
    xbufs = (xb0, xb1, xb2)
    rxbufs = (rxb0, rxb1, rxb2)
    zcbufs = (zcb0, zcb1, zcb2)
    zsbufs = (zsb0, zsb1, zsb2)

    def in_copy(i):
        slot = i % _NBUF
        return [pltpu.make_async_copy(
            xs[v].at[pl.ds(i * _CHUNK, _CHUNK), :], xbufs[v].at[slot], sin.at[slot, v])
            for v in range(3)]

    def out_copy(i):
        slot = i % _NBUF
        return [pltpu.make_async_copy(
            rxbufs[v].at[slot], outs[6 + v].at[pl.ds(i * _CHUNK, _CHUNK), :], srx.at[slot, v])
            for v in range(3)]

    def merge_weights():
        z64 = jnp.zeros((64, 32), _BF)
        z32 = jnp.zeros((32, 64), _BF)
        for v in range(3):
            (Wt, bt, Wc1t, bc1, Wc2t, bc2, Ws1t, bs1, Ws2t, bs2,
             Wdc, bdc, Wds, bds, Wd1, bd1, Wd2, bd2) = wrefs[v * _NW:(v + 1) * _NW]
            wabufs[v][...] = jnp.concatenate(
                [Wc1t[...].T.astype(_BF), Ws1t[...].T.astype(_BF)], axis=1)
            wbbufs[v][...] = jnp.concatenate(
                [jnp.concatenate([Wc2t[...].T.astype(_BF), z64], axis=1),
                 jnp.concatenate([z64, Ws2t[...].T.astype(_BF)], axis=1)], axis=0)
            wcbufs[v][...] = jnp.concatenate(
                [jnp.concatenate([Wdc[...].astype(_BF), z32], axis=1),
                 jnp.concatenate([z32, Wds[...].astype(_BF)], axis=1)], axis=0)
            wtbufs[v][...] = Wt[...].astype(_BF)
            wd1bufs[v][...] = Wd1[...].astype(_BF)
            wd2bufs[v][...] = Wd2[...].astype(_BF)

    def compute(i):
        slot = i % _NBUF
        for v in range(3):
            (Wt, bt, Wc1t, bc1, Wc2t, bc2, Ws1t, bs1, Ws2t, bs2,
             Wdc, bdc, Wds, bds, Wd1, bd1, Wd2, bd2) = wrefs[v * _NW:(v + 1) * _NW]
            x = xbufs[v][slot].astype(_BF)
            z1 = _dot(x, wtbufs[v][...]) + bt[...]
            ba = jnp.concatenate([bc1[...], bs1[...]], axis=1)
            bb = jnp.concatenate([bc2[...], bs2[...]], axis=1)
            bc = jnp.concatenate([bdc[...], bds[...]], axis=1)
            h = jnp.maximum(_dot(z1.astype(_BF), wabufs[v][...]) + ba, 0.0)
            z = _dot(h.astype(_BF), wbbufs[v][...]) + bb
            d = _dot(z.astype(_BF), wcbufs[v][...]) + bc
            g = jnp.maximum(_dot(d.astype(_BF), wd1bufs[v][...]) + bd1[...], 0.0)
            rx = _dot(g.astype(_BF), wd2bufs[v][...]) + bd2[...]
            zt = z.T
            zcbufs[v][:, pl.ds(i * _CHUNK, _CHUNK)] = zt[:32, :]
            zsbufs[v][:, pl.ds(i * _CHUNK, _CHUNK)] = zt[32:, :]
            rxbufs[v][slot] = rx

    w_cps = [pltpu.make_async_copy(whbm[k], wrefs[k], swt.at[k])
             for k in range(3 * _NW)]
    for c in w_cps:
        c.start()
    for i in range(min(_NBUF, _NCHUNK)):
        for c in in_copy(i):
            c.start()
    for c in w_cps:
        c.wait()
    merge_weights()
    for i in range(_NCHUNK):
        for c in in_copy(i):
            c.wait()
        if i >= _NBUF:
            for c in out_copy(i - _NBUF):
                c.wait()
        compute(i)
        for c in out_copy(i):
            c.start()
        if i + _NBUF < _NCHUNK:
            for c in in_copy(i + _NBUF):
                c.start()
    z_cps = []
    for v in range(3):
        z_cps.append(pltpu.make_async_copy(zcbufs[v], outs[v], sz.at[v]))
        z_cps.append(pltpu.make_async_copy(zsbufs[v], outs[3 + v], sz.at[3 + v]))
    for c in z_cps:
        c.start()
    for i in range(max(_NCHUNK - _NBUF, 0), _NCHUNK):
        for c in out_copy(i):
            c.wait()
    for c in z_cps:
        c.wait()


def kernel(x0, x1, x2, trunk_params, content_params, style_params,
           dec_content_params, dec_style_params, dec_trunk_params, status=0):
    xs = (x0, x1, x2)
    weights = []
    for v in range(3):
        (Wt, bt), = trunk_params[v]
        (Wc1, bc1), (Wc2, bc2) = content_params[v]
        (Ws1, bs1), (Ws2, bs2) = style_params[v]
        (Wdc, bdc), = dec_content_params[v]
        (Wds, bds), = dec_style_params[v]
        (Wd1, bd1), (Wd2, bd2) = dec_trunk_params[v]
        weights += [Wt, bt.reshape(1, -1), Wc1.T, bc1.reshape(1, -1),
                    Wc2.T, bc2.reshape(1, -1), Ws1.T, bs1.reshape(1, -1),
                    Ws2.T, bs2.reshape(1, -1), Wdc, bdc.reshape(1, -1),
                    Wds, bds.reshape(1, -1), Wd1, bd1.reshape(1, -1),
                    Wd2, bd2.reshape(1, -1)]

    any_spec = pl.BlockSpec(memory_space=pl.ANY)
    out_shape = (
        [jax.ShapeDtypeStruct((32, _B), jnp.float32) for _ in range(6)]
        + [jax.ShapeDtypeStruct((_B, _VIEW[v]), jnp.float32) for v in range(3)]
    )
    scratch = (
        [pltpu.VMEM(w.shape, jnp.float32) for w in weights]
        + [pltpu.VMEM((_NBUF, _CHUNK, _VIEW[v]), jnp.float32) for v in range(3)]
        + [pltpu.VMEM((_NBUF, _CHUNK, _VIEW[v]), jnp.float32) for v in range(3)]
        + [pltpu.VMEM((32, _B), jnp.float32) for _ in range(6)]
        + [pltpu.VMEM((128, 128), jnp.bfloat16) for _ in range(3)]
        + [pltpu.VMEM((128, 64), jnp.bfloat16) for _ in range(3)]
        + [pltpu.VMEM((64, 128), jnp.bfloat16) for _ in range(3)]
        + [pltpu.VMEM((_VIEW[v], 128), jnp.bfloat16) for v in range(3)]
        + [pltpu.VMEM((128, 128), jnp.bfloat16) for _ in range(3)]
        + [pltpu.VMEM((128, _VIEW[v]), jnp.bfloat16) for v in range(3)]
        + [pltpu.SemaphoreType.DMA((_NBUF, 3))] * 2
        + [pltpu.SemaphoreType.DMA((6,))]
        + [pltpu.SemaphoreType.DMA((3 * _NW,))]
    )
    outs = pl.pallas_call(
        _body,
        in_specs=[any_spec] * (3 + len(weights)),
        out_specs=[any_spec] * 9,
        out_shape=out_shape,
        scratch_shapes=scratch,
        compiler_params=pltpu.CompilerParams(
            disable_bounds_checks=True,
            disable_semaphore_checks=True,
            skip_device_barrier=True,
        ),
    )(*xs, *weights)
    return tuple(o.T for o in outs[:6]) + tuple(outs[6:])


# speedup vs baseline: 1.0489x; 1.0489x over previous
import jax
import jax.numpy as jnp
from jax.experimental import pallas as pl
from jax.experimental.pallas import tpu as pltpu

_B = 16384
_CHUNK = 1024
_NCHUNK = _B // _CHUNK
_NBUF = 4
_VIEW = (128, 256, 512)
_NW = 18


def _dot(a, b):
    return jnp.dot(a, b, preferred_element_type=jnp.float32)


def _body(*refs):
    xs = refs[0:3]
    whbm = refs[3:3 + 3 * _NW]
    outs = refs[3 + 3 * _NW:12 + 3 * _NW]
    wrefs = refs[12 + 3 * _NW:12 + 6 * _NW]
    (xb0, xb1, xb2, rxb0, rxb1, rxb2, zcb0, zcb1, zcb2, zsb0, zsb1, zsb2,
     wa0, wa1, wa2, wb0, wb1, wb2, wc0, wc1, wc2,
     sin, srx, sz, swt) = refs[12 + 6 * _NW:]
    wabufs = (wa0, wa1, wa2)
    wbbufs = (wb0, wb1, wb2)
    wcbufs = (wc0, wc1, wc2)
    xbufs = (xb0, xb1, xb2)
    rxbufs = (rxb0, rxb1, rxb2)
    zcbufs = (zcb0, zcb1, zcb2)
    zsbufs = (zsb0, zsb1, zsb2)

    def in_copy(i):
        slot = i % _NBUF
        return [pltpu.make_async_copy(
            xs[v].at[pl.ds(i * _CHUNK, _CHUNK), :], xbufs[v].at[slot], sin.at[slot, v])
            for v in range(3)]

    def out_copy(i):
        slot = i % _NBUF
        return [pltpu.make_async_copy(
            rxbufs[v].at[slot], outs[6 + v].at[pl.ds(i * _CHUNK, _CHUNK), :], srx.at[slot, v])
            for v in range(3)]

    def merge_weights():
        z64 = jnp.zeros((64, 32), jnp.float32)
        z32 = jnp.zeros((32, 64), jnp.float32)
        for v in range(3):
            (Wt, bt, Wc1t, bc1, Wc2t, bc2, Ws1t, bs1, Ws2t, bs2,
             Wdc, bdc, Wds, bds, Wd1, bd1, Wd2, bd2) = wrefs[v * _NW:(v + 1) * _NW]
            wabufs[v][...] = jnp.concatenate([Wc1t[...].T, Ws1t[...].T], axis=1)
            wbbufs[v][...] = jnp.concatenate(
                [jnp.concatenate([Wc2t[...].T, z64], axis=1),
                 jnp.concatenate([z64, Ws2t[...].T], axis=1)], axis=0)
            wcbufs[v][...] = jnp.concatenate(
                [jnp.concatenate([Wdc[...], z32], axis=1),
                 jnp.concatenate([z32, Wds[...]], axis=1)], axis=0)

    def compute(i):
        slot = i % _NBUF
        for v in range(3):
            (Wt, bt, Wc1t, bc1, Wc2t, bc2, Ws1t, bs1, Ws2t, bs2,
             Wdc, bdc, Wds, bds, Wd1, bd1, Wd2, bd2) = wrefs[v * _NW:(v + 1) * _NW]
            x = xbufs[v][slot]
            z1 = _dot(x, Wt[...]) + bt[...]
            ba = jnp.concatenate([bc1[...], bs1[...]], axis=1)
            bb = jnp.concatenate([bc2[...], bs2[...]], axis=1)
            bc = jnp.concatenate([bdc[...], bds[...]], axis=1)
            h = jnp.maximum(_dot(z1, wabufs[v][...]) + ba, 0.0)
            z = _dot(h, wbbufs[v][...]) + bb
            d = _dot(z, wcbufs[v][...]) + bc
            g = jnp.maximum(_dot(d, Wd1[...]) + bd1[...], 0.0)
            rx = _dot(g, Wd2[...]) + bd2[...]
            zt = z.T
            zcbufs[v][:, pl.ds(i * _CHUNK, _CHUNK)] = zt[:32, :]
            zsbufs[v][:, pl.ds(i * _CHUNK, _CHUNK)] = zt[32:, :]
            rxbufs[v][slot] = rx

    HB = _B // 2

    def z_flush(half):
        lo = half * HB
        cps = []
        for v in range(3):
            cps.append(pltpu.make_async_copy(
                zcbufs[v].at[:, pl.ds(lo, HB)], outs[v].at[:, pl.ds(lo, HB)],
                sz.at[half, v]))
            cps.append(pltpu.make_async_copy(
                zsbufs[v].at[:, pl.ds(lo, HB)], outs[3 + v].at[:, pl.ds(lo, HB)],
                sz.at[half, 3 + v]))
        return cps

    w_cps = [pltpu.make_async_copy(whbm[k], wrefs[k], swt.at[k])
             for k in range(3 * _NW)]
    for c in w_cps:
        c.start()
    for i in range(min(_NBUF, _NCHUNK)):
        for c in in_copy(i):
            c.start()
    for c in w_cps:
        c.wait()
    merge_weights()
    for i in range(_NCHUNK):
        for c in in_copy(i):
            c.wait()
        if i >= _NBUF:
            for c in out_copy(i - _NBUF):
                c.wait()
        compute(i)
        for c in out_copy(i):
            c.start()
        if i == _NCHUNK // 2 - 1:
            for c in z_flush(0):
                c.start()
        if i + _NBUF < _NCHUNK:
            for c in in_copy(i + _NBUF):
                c.start()
    for c in z_flush(1):
        c.start()
    for i in range(max(_NCHUNK - _NBUF, 0), _NCHUNK):
        for c in out_copy(i):
            c.wait()
    for h in (0, 1):
        for c in z_flush(h):
            c.wait()


def kernel(x0, x1, x2, trunk_params, content_params, style_params,
           dec_content_params, dec_style_params, dec_trunk_params, status=0):
    xs = (x0, x1, x2)
    weights = []
    for v in range(3):
        (Wt, bt), = trunk_params[v]
        (Wc1, bc1), (Wc2, bc2) = content_params[v]
        (Ws1, bs1), (Ws2, bs2) = style_params[v]
        (Wdc, bdc), = dec_content_params[v]
        (Wds, bds), = dec_style_params[v]
        (Wd1, bd1), (Wd2, bd2) = dec_trunk_params[v]
        weights += [Wt, bt.reshape(1, -1), Wc1.T, bc1.reshape(1, -1),
                    Wc2.T, bc2.reshape(1, -1), Ws1.T, bs1.reshape(1, -1),
                    Ws2.T, bs2.reshape(1, -1), Wdc, bdc.reshape(1, -1),
                    Wds, bds.reshape(1, -1), Wd1, bd1.reshape(1, -1),
                    Wd2, bd2.reshape(1, -1)]

    any_spec = pl.BlockSpec(memory_space=pl.ANY)
    out_shape = (
        [jax.ShapeDtypeStruct((32, _B), jnp.float32) for _ in range(6)]
        + [jax.ShapeDtypeStruct((_B, _VIEW[v]), jnp.float32) for v in range(3)]
    )
    scratch = (
        [pltpu.VMEM(w.shape, jnp.float32) for w in weights]
        + [pltpu.VMEM((_NBUF, _CHUNK, _VIEW[v]), jnp.float32) for v in range(3)]
        + [pltpu.VMEM((_NBUF, _CHUNK, _VIEW[v]), jnp.float32) for v in range(3)]
        + [pltpu.VMEM((32, _B), jnp.float32) for _ in range(6)]
        + [pltpu.VMEM((128, 128), jnp.float32) for _ in range(3)]
        + [pltpu.VMEM((128, 64), jnp.float32) for _ in range(3)]
        + [pltpu.VMEM((64, 128), jnp.float32) for _ in range(3)]
        + [pltpu.SemaphoreType.DMA((_NBUF, 3))] * 2
        + [pltpu.SemaphoreType.DMA((2, 6))]
        + [pltpu.SemaphoreType.DMA((3 * _NW,))]
    )
    outs = pl.pallas_call(
        _body,
        in_specs=[any_spec] * (3 + len(weights)),
        out_specs=[any_spec] * 9,
        out_shape=out_shape,
        scratch_shapes=scratch,
        compiler_params=pltpu.CompilerParams(
            disable_bounds_checks=True,
            disable_semaphore_checks=True,
            skip_device_barrier=True,
        ),
    )(*xs, *weights)
    return tuple(o.T for o in outs[:6]) + tuple(outs[6:])


# speedup vs baseline: 1.2318x; 1.1744x over previous
import jax
import jax.numpy as jnp
from jax.experimental import pallas as pl
from jax.experimental.pallas import tpu as pltpu

_B = 16384
_CHUNK = 2048
_NCHUNK = _B // _CHUNK
_NBUF = 3
_VIEW = (128, 256, 512)
_NW = 18


def _dot(a, b):
    return jnp.dot(a, b, preferred_element_type=jnp.float32)


def _body(*refs):
    xs = refs[0:3]
    whbm = refs[3:3 + 3 * _NW]
    outs = refs[3 + 3 * _NW:12 + 3 * _NW]
    wrefs = refs[12 + 3 * _NW:12 + 6 * _NW]
    (xb0, xb1, xb2, rxb0, rxb1, rxb2, zcb0, zcb1, zcb2, zsb0, zsb1, zsb2,
     wa0, wa1, wa2, wb0, wb1, wb2, wc0, wc1, wc2,
     sin, srx, sz, swt) = refs[12 + 6 * _NW:]
    wabufs = (wa0, wa1, wa2)
    wbbufs = (wb0, wb1, wb2)
    wcbufs = (wc0, wc1, wc2)
    xbufs = (xb0, xb1, xb2)
    rxbufs = (rxb0, rxb1, rxb2)
    zcbufs = (zcb0, zcb1, zcb2)
    zsbufs = (zsb0, zsb1, zsb2)

    def in_copy(i):
        slot = i % _NBUF
        return [pltpu.make_async_copy(
            xs[v].at[pl.ds(i * _CHUNK, _CHUNK), :], xbufs[v].at[slot], sin.at[slot, v])
            for v in range(3)]

    def out_copy(i):
        slot = i % _NBUF
        return [pltpu.make_async_copy(
            rxbufs[v].at[slot], outs[6 + v].at[pl.ds(i * _CHUNK, _CHUNK), :], srx.at[slot, v])
            for v in range(3)]

    def merge_weights():
        z64 = jnp.zeros((64, 32), jnp.float32)
        z32 = jnp.zeros((32, 64), jnp.float32)
        for v in range(3):
            (Wt, bt, Wc1t, bc1, Wc2t, bc2, Ws1t, bs1, Ws2t, bs2,
             Wdc, bdc, Wds, bds, Wd1, bd1, Wd2, bd2) = wrefs[v * _NW:(v + 1) * _NW]
            wabufs[v][...] = jnp.concatenate([Wc1t[...].T, Ws1t[...].T], axis=1)
            wbbufs[v][...] = jnp.concatenate(
                [jnp.concatenate([Wc2t[...].T, z64], axis=1),
                 jnp.concatenate([z64, Ws2t[...].T], axis=1)], axis=0)
            wcbufs[v][...] = jnp.concatenate(
                [jnp.concatenate([Wdc[...], z32], axis=1),
                 jnp.concatenate([z32, Wds[...]], axis=1)], axis=0)

    def compute(i):
        slot = i % _NBUF
        for v in range(3):
            (Wt, bt, Wc1t, bc1, Wc2t, bc2, Ws1t, bs1, Ws2t, bs2,
             Wdc, bdc, Wds, bds, Wd1, bd1, Wd2, bd2) = wrefs[v * _NW:(v + 1) * _NW]
            x = xbufs[v][slot]
            z1 = _dot(x, Wt[...]) + bt[...]
            ba = jnp.concatenate([bc1[...], bs1[...]], axis=1)
            bb = jnp.concatenate([bc2[...], bs2[...]], axis=1)
            bc = jnp.concatenate([bdc[...], bds[...]], axis=1)
            h = jnp.maximum(_dot(z1, wabufs[v][...]) + ba, 0.0)
            z = _dot(h, wbbufs[v][...]) + bb
            d = _dot(z, wcbufs[v][...]) + bc
            g = jnp.maximum(_dot(d, Wd1[...]) + bd1[...], 0.0)
            rx = _dot(g, Wd2[...]) + bd2[...]
            zt = z.T
            zcbufs[v][:, pl.ds(i * _CHUNK, _CHUNK)] = zt[:32, :]
            zsbufs[v][:, pl.ds(i * _CHUNK, _CHUNK)] = zt[32:, :]
            rxbufs[v][slot] = rx

    HB = _B // 2

    def z_flush(half):
        lo = half * HB
        cps = []
        for v in range(3):
            cps.append(pltpu.make_async_copy(
                zcbufs[v].at[:, pl.ds(lo, HB)], outs[v].at[:, pl.ds(lo, HB)],
                sz.at[half, v]))
            cps.append(pltpu.make_async_copy(
                zsbufs[v].at[:, pl.ds(lo, HB)], outs[3 + v].at[:, pl.ds(lo, HB)],
                sz.at[half, 3 + v]))
        return cps

    w_cps = [pltpu.make_async_copy(whbm[k], wrefs[k], swt.at[k])
             for k in range(3 * _NW)]
    for c in w_cps:
        c.start()
    for i in range(min(_NBUF, _NCHUNK)):
        for c in in_copy(i):
            c.start()
    for c in w_cps:
        c.wait()
    merge_weights()
    for i in range(_NCHUNK):
        for c in in_copy(i):
            c.wait()
        if i >= _NBUF:
            for c in out_copy(i - _NBUF):
                c.wait()
        compute(i)
        for c in out_copy(i):
            c.start()
        if i == _NCHUNK // 2 - 1:
            for c in z_flush(0):
                c.start()
        if i + _NBUF < _NCHUNK:
            for c in in_copy(i + _NBUF):
                c.start()
    for c in z_flush(1):
        c.start()
    for i in range(max(_NCHUNK - _NBUF, 0), _NCHUNK):
        for c in out_copy(i):
            c.wait()
    for h in (0, 1):
        for c in z_flush(h):
            c.wait()


def kernel(x0, x1, x2, trunk_params, content_params, style_params,
           dec_content_params, dec_style_params, dec_trunk_params, status=0):
    xs = (x0, x1, x2)
    weights = []
    for v in range(3):
        (Wt, bt), = trunk_params[v]
        (Wc1, bc1), (Wc2, bc2) = content_params[v]
        (Ws1, bs1), (Ws2, bs2) = style_params[v]
        (Wdc, bdc), = dec_content_params[v]
        (Wds, bds), = dec_style_params[v]
        (Wd1, bd1), (Wd2, bd2) = dec_trunk_params[v]
        weights += [Wt, bt.reshape(1, -1), Wc1.T, bc1.reshape(1, -1),
                    Wc2.T, bc2.reshape(1, -1), Ws1.T, bs1.reshape(1, -1),
                    Ws2.T, bs2.reshape(1, -1), Wdc, bdc.reshape(1, -1),
                    Wds, bds.reshape(1, -1), Wd1, bd1.reshape(1, -1),
                    Wd2, bd2.reshape(1, -1)]

    any_spec = pl.BlockSpec(memory_space=pl.ANY)
    out_shape = (
        [jax.ShapeDtypeStruct((32, _B), jnp.float32) for _ in range(6)]
        + [jax.ShapeDtypeStruct((_B, _VIEW[v]), jnp.float32) for v in range(3)]
    )
    scratch = (
        [pltpu.VMEM(w.shape, jnp.float32) for w in weights]
        + [pltpu.VMEM((_NBUF, _CHUNK, _VIEW[v]), jnp.float32) for v in range(3)]
        + [pltpu.VMEM((_NBUF, _CHUNK, _VIEW[v]), jnp.float32) for v in range(3)]
        + [pltpu.VMEM((32, _B), jnp.float32) for _ in range(6)]
        + [pltpu.VMEM((128, 128), jnp.float32) for _ in range(3)]
        + [pltpu.VMEM((128, 64), jnp.float32) for _ in range(3)]
        + [pltpu.VMEM((64, 128), jnp.float32) for _ in range(3)]
        + [pltpu.SemaphoreType.DMA((_NBUF, 3))] * 2
        + [pltpu.SemaphoreType.DMA((2, 6))]
        + [pltpu.SemaphoreType.DMA((3 * _NW,))]
    )
    outs = pl.pallas_call(
        _body,
        in_specs=[any_spec] * (3 + len(weights)),
        out_specs=[any_spec] * 9,
        out_shape=out_shape,
        scratch_shapes=scratch,
        compiler_params=pltpu.CompilerParams(
            disable_bounds_checks=True,
            vmem_limit_bytes=64 * 1024 * 1024,
            disable_semaphore_checks=True,
            skip_device_barrier=True,
        ),
    )(*xs, *weights)
    return tuple(o.T for o in outs[:6]) + tuple(outs[6:])
